# baseline (speedup 1.0000x reference)
import functools

import jax
import jax.numpy as jnp
from jax import lax
from jax.experimental import pallas as pl
from jax.experimental.pallas import tpu as pltpu
from jax.experimental.pallas import tpu_sc as plsc

B = 16384
F = 26
V = 100000
D = 16
N_DENSE = 13
SP = F * D

NW = 32
ROWS_PER_W = SP // NW
OQ = B // 4
UNROLL = 16


def _gather_body(idxT_hbm, table_hbm, out_hbm, idx_v, row_v, out_v, ws0, ws1):
    wid = lax.axis_index("s") * 2 + lax.axis_index("c")
    r0 = wid * ROWS_PER_W

    def do_row(j, f_prev):
        r = r0 + j
        f = r // D

        @pl.when(f != f_prev)
        def _stage_idx():
            pltpu.sync_copy(idxT_hbm.at[f], idx_v)

        pltpu.sync_copy(table_hbm.at[r], row_v)

        sems = (ws0, ws1)
        for q in range(4):
            b = q % 2
            sem = sems[b]
            if q >= 2:
                pltpu.make_async_copy(
                    out_v.at[b], out_hbm.at[r, pl.ds((q - 2) * OQ, OQ)], sem
                ).wait()
            else:

                @pl.when(j > 0)
                def _drain():
                    pltpu.make_async_copy(
                        out_v.at[b], out_hbm.at[r - 1, pl.ds((q + 2) * OQ, OQ)], sem
                    ).wait()

            def gblk(i, carry3):
                for u in range(UNROLL):
                    off = (i * UNROLL + u) * 16
                    iv = idx_v[pl.ds(q * OQ + off, 16)]
                    out_v[b, pl.ds(off, 16)] = plsc.load_gather(row_v, [iv])
                return carry3

            lax.fori_loop(0, OQ // (16 * UNROLL), gblk, 0)
            pltpu.async_copy(out_v.at[b], out_hbm.at[r, pl.ds(q * OQ, OQ)], sem)
        return f

    lax.fori_loop(0, ROWS_PER_W, do_row, jnp.int32(-1))
    r_last = r0 + ROWS_PER_W - 1
    pltpu.make_async_copy(out_v.at[0], out_hbm.at[r_last, pl.ds(2 * OQ, OQ)], ws0).wait()
    pltpu.make_async_copy(out_v.at[1], out_hbm.at[r_last, pl.ds(3 * OQ, OQ)], ws1).wait()


@functools.cache
def _gather():
    return pl.kernel(
        _gather_body,
        out_type=jax.ShapeDtypeStruct((SP, B), jnp.float32),
        mesh=plsc.VectorSubcoreMesh(core_axis_name="c", subcore_axis_name="s"),
        scratch_types=[
            pltpu.VMEM((B,), jnp.int32),
            pltpu.VMEM((V,), jnp.float32),
            pltpu.VMEM((2, OQ), jnp.float32),
            pltpu.SemaphoreType.DMA,
            pltpu.SemaphoreType.DMA,
        ],
        compiler_params=pltpu.CompilerParams(needs_layout_passes=False),
    )


BB = 2048


def _tc_body(xs_ref, xd_ref, w1_ref, b1_ref, w2_ref, b2_ref,
             w3t_ref, b3_ref, wdt_ref, bd_ref, wfmt_ref, bfm_ref,
             o_ref):
    f32 = jnp.float32
    prec = lax.Precision.DEFAULT

    def dott(a, b):
        return lax.dot_general(a, b, (((0,), (0,)), ((), ())),
                               preferred_element_type=f32, precision=prec)

    def dotn(a, b):
        return lax.dot_general(a, b, (((1,), (0,)), ((), ())),
                               preferred_element_type=f32, precision=prec)

    xs = xs_ref[...]
    xd = xd_ref[...]
    ci = lax.broadcasted_iota(jnp.int32, (D, SP), 0)
    cj = lax.broadcasted_iota(jnp.int32, (D, SP), 1)
    sel = jnp.where((cj % D) == ci, 1.0, 0.0).astype(f32)
    s1 = dotn(sel, xs)
    s2 = dotn(sel, xs * xs)
    fm2 = 0.5 * jnp.sum(s1 * s1 - s2, axis=0, keepdims=True)
    wfmt = wfmt_ref[...]
    fm1 = dotn(wfmt[:, :SP], xs) + dotn(wfmt[:, SP:], xd) + bfm_ref[...]
    w1 = w1_ref[...]
    h = jnp.maximum(dott(w1[:SP], xs) + dott(w1[SP:], xd) + b1_ref[...], 0.0)
    h = jnp.maximum(dott(w2_ref[...], h) + b2_ref[...], 0.0)
    h = jnp.maximum(dotn(w3t_ref[...], h) + b3_ref[...], 0.0)
    dnn = dotn(wdt_ref[...], h) + bd_ref[...]
    o_ref[...] = jax.nn.sigmoid(fm1 + fm2 + dnn)


def _full(shape):
    return pl.BlockSpec(shape, lambda i: tuple(0 for _ in shape))


_tc_call = pl.pallas_call(
    _tc_body,
    grid=(B // BB,),
    in_specs=[
        pl.BlockSpec((SP, BB), lambda i: (0, i)),
        pl.BlockSpec((N_DENSE, BB), lambda i: (0, i)),
        _full((SP + N_DENSE, 256)),
        _full((256, 1)),
        _full((256, 128)),
        _full((128, 1)),
        _full((64, 128)),
        _full((64, 1)),
        _full((1, 64)),
        _full((1, 1)),
        _full((1, SP + N_DENSE)),
        _full((1, 1)),
    ],
    out_specs=pl.BlockSpec((1, BB), lambda i: (0, i)),
    out_shape=jax.ShapeDtypeStruct((1, B), jnp.float32),
)


def kernel(dense_input, sparse_input, embed_tables, W_fm, b_fm,
           W1, b1, W2, b2, W3, b3, Wd, bd):
    tableT = embed_tables.transpose(0, 2, 1).reshape(SP, V)
    idxT = sparse_input.T
    xsT = _gather()(idxT, tableT)
    outT = _tc_call(
        xsT, dense_input.T,
        W1, b1.reshape(-1, 1),
        W2, b2.reshape(-1, 1),
        W3.T, b3.reshape(-1, 1),
        Wd.T, bd.reshape(1, 1),
        W_fm.T, b_fm.reshape(1, 1),
    )
    return outT.reshape(B)


# speedup vs baseline: 1.3444x; 1.3444x over previous
import functools

import jax
import jax.numpy as jnp
from jax import lax
from jax.experimental import pallas as pl
from jax.experimental.pallas import tpu as pltpu
from jax.experimental.pallas import tpu_sc as plsc

B = 16384
F = 26
V = 100000
D = 16
N_DENSE = 13
SP = F * D

NW = 32
ROWS_PER_W = SP // NW
OH = B // 2
UNROLL = 8


def _gather_body(idxT_hbm, table_hbm, out_hbm, idx_v, row_v, out_v):
    wid = lax.axis_index("s") * 2 + lax.axis_index("c")
    r0 = wid * ROWS_PER_W

    def do_row(j, f_prev):
        r = r0 + j
        f = r // D

        @pl.when(f != f_prev)
        def _stage_idx():
            pltpu.sync_copy(idxT_hbm.at[f], idx_v)

        pltpu.sync_copy(table_hbm.at[r], row_v)

        def do_half(h, carry2):
            def gblk(i, carry3):
                for u in range(UNROLL):
                    off = (i * UNROLL + u) * 16
                    iv = idx_v[pl.ds(h * OH + off, 16)]
                    out_v[pl.ds(off, 16)] = plsc.load_gather(row_v, [iv])
                return carry3

            lax.fori_loop(0, OH // (16 * UNROLL), gblk, 0)
            pltpu.sync_copy(out_v, out_hbm.at[r, pl.ds(h * OH, OH)])
            return carry2

        lax.fori_loop(0, 2, do_half, 0)
        return f

    lax.fori_loop(0, ROWS_PER_W, do_row, jnp.int32(-1))


@functools.cache
def _gather():
    return pl.kernel(
        _gather_body,
        out_type=jax.ShapeDtypeStruct((SP, B), jnp.float32),
        mesh=plsc.VectorSubcoreMesh(core_axis_name="c", subcore_axis_name="s"),
        scratch_types=[
            pltpu.VMEM((B,), jnp.int32),
            pltpu.VMEM((V,), jnp.float32),
            pltpu.VMEM((OH,), jnp.float32),
        ],
        compiler_params=pltpu.CompilerParams(needs_layout_passes=False),
    )


BB = 2048


def _tc_body(xs_ref, xd_ref, w1_ref, b1_ref, w2_ref, b2_ref,
             w3t_ref, b3_ref, wdt_ref, bd_ref, wfmt_ref, bfm_ref,
             o_ref):
    f32 = jnp.float32
    prec = lax.Precision.DEFAULT

    def dott(a, b):
        return lax.dot_general(a, b, (((0,), (0,)), ((), ())),
                               preferred_element_type=f32, precision=prec)

    def dotn(a, b):
        return lax.dot_general(a, b, (((1,), (0,)), ((), ())),
                               preferred_element_type=f32, precision=prec)

    xs = xs_ref[...]
    xd = xd_ref[...]
    ci = lax.broadcasted_iota(jnp.int32, (D, SP), 0)
    cj = lax.broadcasted_iota(jnp.int32, (D, SP), 1)
    sel = jnp.where((cj % D) == ci, 1.0, 0.0).astype(f32)
    s1 = dotn(sel, xs)
    s2 = dotn(sel, xs * xs)
    fm2 = 0.5 * jnp.sum(s1 * s1 - s2, axis=0, keepdims=True)
    wfmt = wfmt_ref[...]
    fm1 = dotn(wfmt[:, :SP], xs) + dotn(wfmt[:, SP:], xd) + bfm_ref[...]
    w1 = w1_ref[...]
    h = jnp.maximum(dott(w1[:SP], xs) + dott(w1[SP:], xd) + b1_ref[...], 0.0)
    h = jnp.maximum(dott(w2_ref[...], h) + b2_ref[...], 0.0)
    h = jnp.maximum(dotn(w3t_ref[...], h) + b3_ref[...], 0.0)
    dnn = dotn(wdt_ref[...], h) + bd_ref[...]
    o_ref[...] = jax.nn.sigmoid(fm1 + fm2 + dnn)


def _full(shape):
    return pl.BlockSpec(shape, lambda i: tuple(0 for _ in shape))


_tc_call = pl.pallas_call(
    _tc_body,
    grid=(B // BB,),
    in_specs=[
        pl.BlockSpec((SP, BB), lambda i: (0, i)),
        pl.BlockSpec((N_DENSE, BB), lambda i: (0, i)),
        _full((SP + N_DENSE, 256)),
        _full((256, 1)),
        _full((256, 128)),
        _full((128, 1)),
        _full((64, 128)),
        _full((64, 1)),
        _full((1, 64)),
        _full((1, 1)),
        _full((1, SP + N_DENSE)),
        _full((1, 1)),
    ],
    out_specs=pl.BlockSpec((1, BB), lambda i: (0, i)),
    out_shape=jax.ShapeDtypeStruct((1, B), jnp.float32),
)


def kernel(dense_input, sparse_input, embed_tables, W_fm, b_fm,
           W1, b1, W2, b2, W3, b3, Wd, bd):
    tableT = embed_tables.transpose(0, 2, 1).reshape(SP, V)
    idxT = sparse_input.T
    xsT = _gather()(idxT, tableT)
    outT = _tc_call(
        xsT, dense_input.T,
        W1, b1.reshape(-1, 1),
        W2, b2.reshape(-1, 1),
        W3.T, b3.reshape(-1, 1),
        Wd.T, bd.reshape(1, 1),
        W_fm.T, b_fm.reshape(1, 1),
    )
    return outT.reshape(B)


# speedup vs baseline: 1.7255x; 1.2835x over previous
import functools

import jax
import jax.numpy as jnp
from jax import lax
from jax.experimental import pallas as pl
from jax.experimental.pallas import tpu as pltpu
from jax.experimental.pallas import tpu_sc as plsc

B = 16384
F = 26
V = 100000
D = 16
N_DENSE = 13
SP = F * D

NW = 32
ROWS_PER_W = SP // NW
OH = B // 2
UNROLL = 8


def _gather_body(idxT_hbm, table_hbm, out_hbm, idx_v, row_v, out_v):
    wid = lax.axis_index("s") * 2 + lax.axis_index("c")
    r0 = wid * ROWS_PER_W

    def do_row(j, f_prev):
        r = r0 + j
        f = r // D

        @pl.when(f != f_prev)
        def _stage_idx():
            pltpu.sync_copy(idxT_hbm.at[f], idx_v)

        pltpu.sync_copy(table_hbm.at[r], row_v)

        def do_half(h, carry2):
            @plsc.parallel_loop(0, OH, step=16, unroll=UNROLL)
            def gblk(i):
                iv = idx_v[pl.ds(h * OH + i, 16)]
                out_v[pl.ds(i, 16)] = plsc.load_gather(row_v, [iv])

            pltpu.sync_copy(out_v, out_hbm.at[r, pl.ds(h * OH, OH)])
            return carry2

        lax.fori_loop(0, 2, do_half, 0)
        return f

    lax.fori_loop(0, ROWS_PER_W, do_row, jnp.int32(-1))


@functools.cache
def _gather():
    return pl.kernel(
        _gather_body,
        out_type=jax.ShapeDtypeStruct((SP, B), jnp.float32),
        mesh=plsc.VectorSubcoreMesh(core_axis_name="c", subcore_axis_name="s"),
        scratch_types=[
            pltpu.VMEM((B,), jnp.int32),
            pltpu.VMEM((V,), jnp.float32),
            pltpu.VMEM((OH,), jnp.float32),
        ],
        compiler_params=pltpu.CompilerParams(needs_layout_passes=False),
    )


BB = 2048


def _tc_body(xs_ref, xd_ref, w1_ref, b1_ref, w2_ref, b2_ref,
             w3t_ref, b3_ref, wdt_ref, bd_ref, wfmt_ref, bfm_ref,
             o_ref):
    f32 = jnp.float32
    prec = lax.Precision.DEFAULT

    def dott(a, b):
        return lax.dot_general(a, b, (((0,), (0,)), ((), ())),
                               preferred_element_type=f32, precision=prec)

    def dotn(a, b):
        return lax.dot_general(a, b, (((1,), (0,)), ((), ())),
                               preferred_element_type=f32, precision=prec)

    xs = xs_ref[...]
    xd = xd_ref[...]
    ci = lax.broadcasted_iota(jnp.int32, (D, SP), 0)
    cj = lax.broadcasted_iota(jnp.int32, (D, SP), 1)
    sel = jnp.where((cj % D) == ci, 1.0, 0.0).astype(f32)
    s1 = dotn(sel, xs)
    s2 = dotn(sel, xs * xs)
    fm2 = 0.5 * jnp.sum(s1 * s1 - s2, axis=0, keepdims=True)
    wfmt = wfmt_ref[...]
    fm1 = dotn(wfmt[:, :SP], xs) + dotn(wfmt[:, SP:], xd) + bfm_ref[...]
    w1 = w1_ref[...]
    h = jnp.maximum(dott(w1[:SP], xs) + dott(w1[SP:], xd) + b1_ref[...], 0.0)
    h = jnp.maximum(dott(w2_ref[...], h) + b2_ref[...], 0.0)
    h = jnp.maximum(dotn(w3t_ref[...], h) + b3_ref[...], 0.0)
    dnn = dotn(wdt_ref[...], h) + bd_ref[...]
    o_ref[...] = jax.nn.sigmoid(fm1 + fm2 + dnn)


def _full(shape):
    return pl.BlockSpec(shape, lambda i: tuple(0 for _ in shape))


_tc_call = pl.pallas_call(
    _tc_body,
    grid=(B // BB,),
    in_specs=[
        pl.BlockSpec((SP, BB), lambda i: (0, i)),
        pl.BlockSpec((N_DENSE, BB), lambda i: (0, i)),
        _full((SP + N_DENSE, 256)),
        _full((256, 1)),
        _full((256, 128)),
        _full((128, 1)),
        _full((64, 128)),
        _full((64, 1)),
        _full((1, 64)),
        _full((1, 1)),
        _full((1, SP + N_DENSE)),
        _full((1, 1)),
    ],
    out_specs=pl.BlockSpec((1, BB), lambda i: (0, i)),
    out_shape=jax.ShapeDtypeStruct((1, B), jnp.float32),
)


def kernel(dense_input, sparse_input, embed_tables, W_fm, b_fm,
           W1, b1, W2, b2, W3, b3, Wd, bd):
    tableT = embed_tables.transpose(0, 2, 1).reshape(SP, V)
    idxT = sparse_input.T
    xsT = _gather()(idxT, tableT)
    outT = _tc_call(
        xsT, dense_input.T,
        W1, b1.reshape(-1, 1),
        W2, b2.reshape(-1, 1),
        W3.T, b3.reshape(-1, 1),
        Wd.T, bd.reshape(1, 1),
        W_fm.T, b_fm.reshape(1, 1),
    )
    return outT.reshape(B)


# speedup vs baseline: 1.7296x; 1.0024x over previous
import functools

import jax
import jax.numpy as jnp
from jax import lax
from jax.experimental import pallas as pl
from jax.experimental.pallas import tpu as pltpu
from jax.experimental.pallas import tpu_sc as plsc

B = 16384
F = 26
V = 100000
D = 16
N_DENSE = 13
SP = F * D

NW = 32
ROWS_PER_W = SP // NW
OH = B // 2
UNROLL = 16


def _gather_body(idxT_hbm, table_hbm, out_hbm, idx_v, row_v, out_v):
    wid = lax.axis_index("s") * 2 + lax.axis_index("c")
    r0 = wid * ROWS_PER_W

    def do_row(j, f_prev):
        r = r0 + j
        f = r // D

        @pl.when(f != f_prev)
        def _stage_idx():
            pltpu.sync_copy(idxT_hbm.at[f], idx_v)

        pltpu.sync_copy(table_hbm.at[r], row_v)

        def do_half(h, carry2):
            @plsc.parallel_loop(0, OH, step=16, unroll=UNROLL)
            def gblk(i):
                iv = idx_v[pl.ds(h * OH + i, 16)]
                out_v[pl.ds(i, 16)] = plsc.load_gather(row_v, [iv])

            pltpu.sync_copy(out_v, out_hbm.at[r, pl.ds(h * OH, OH)])
            return carry2

        lax.fori_loop(0, 2, do_half, 0)
        return f

    lax.fori_loop(0, ROWS_PER_W, do_row, jnp.int32(-1))


@functools.cache
def _gather():
    return pl.kernel(
        _gather_body,
        out_type=jax.ShapeDtypeStruct((SP, B), jnp.float32),
        mesh=plsc.VectorSubcoreMesh(core_axis_name="c", subcore_axis_name="s"),
        scratch_types=[
            pltpu.VMEM((B,), jnp.int32),
            pltpu.VMEM((V,), jnp.float32),
            pltpu.VMEM((OH,), jnp.float32),
        ],
        compiler_params=pltpu.CompilerParams(needs_layout_passes=False),
    )


BB = 2048


def _tc_body(xs_ref, xd_ref, w1_ref, b1_ref, w2_ref, b2_ref,
             w3t_ref, b3_ref, wdt_ref, bd_ref, wfmt_ref, bfm_ref,
             o_ref):
    f32 = jnp.float32
    prec = lax.Precision.DEFAULT

    def dott(a, b):
        return lax.dot_general(a, b, (((0,), (0,)), ((), ())),
                               preferred_element_type=f32, precision=prec)

    def dotn(a, b):
        return lax.dot_general(a, b, (((1,), (0,)), ((), ())),
                               preferred_element_type=f32, precision=prec)

    xs = xs_ref[...]
    xd = xd_ref[...]
    ci = lax.broadcasted_iota(jnp.int32, (D, SP), 0)
    cj = lax.broadcasted_iota(jnp.int32, (D, SP), 1)
    sel = jnp.where((cj % D) == ci, 1.0, 0.0).astype(f32)
    s1 = dotn(sel, xs)
    s2 = dotn(sel, xs * xs)
    fm2 = 0.5 * jnp.sum(s1 * s1 - s2, axis=0, keepdims=True)
    wfmt = wfmt_ref[...]
    fm1 = dotn(wfmt[:, :SP], xs) + dotn(wfmt[:, SP:], xd) + bfm_ref[...]
    w1 = w1_ref[...]
    h = jnp.maximum(dott(w1[:SP], xs) + dott(w1[SP:], xd) + b1_ref[...], 0.0)
    h = jnp.maximum(dott(w2_ref[...], h) + b2_ref[...], 0.0)
    h = jnp.maximum(dotn(w3t_ref[...], h) + b3_ref[...], 0.0)
    dnn = dotn(wdt_ref[...], h) + bd_ref[...]
    o_ref[...] = jax.nn.sigmoid(fm1 + fm2 + dnn)


def _full(shape):
    return pl.BlockSpec(shape, lambda i: tuple(0 for _ in shape))


_tc_call = pl.pallas_call(
    _tc_body,
    grid=(B // BB,),
    in_specs=[
        pl.BlockSpec((SP, BB), lambda i: (0, i)),
        pl.BlockSpec((N_DENSE, BB), lambda i: (0, i)),
        _full((SP + N_DENSE, 256)),
        _full((256, 1)),
        _full((256, 128)),
        _full((128, 1)),
        _full((64, 128)),
        _full((64, 1)),
        _full((1, 64)),
        _full((1, 1)),
        _full((1, SP + N_DENSE)),
        _full((1, 1)),
    ],
    out_specs=pl.BlockSpec((1, BB), lambda i: (0, i)),
    out_shape=jax.ShapeDtypeStruct((1, B), jnp.float32),
)


def kernel(dense_input, sparse_input, embed_tables, W_fm, b_fm,
           W1, b1, W2, b2, W3, b3, Wd, bd):
    tableT = embed_tables.transpose(0, 2, 1).reshape(SP, V)
    idxT = sparse_input.T
    xsT = _gather()(idxT, tableT)
    outT = _tc_call(
        xsT, dense_input.T,
        W1, b1.reshape(-1, 1),
        W2, b2.reshape(-1, 1),
        W3.T, b3.reshape(-1, 1),
        Wd.T, bd.reshape(1, 1),
        W_fm.T, b_fm.reshape(1, 1),
    )
    return outT.reshape(B)


# speedup vs baseline: 1.7492x; 1.0113x over previous
import functools

import jax
import jax.numpy as jnp
from jax import lax
from jax.experimental import pallas as pl
from jax.experimental.pallas import tpu as pltpu
from jax.experimental.pallas import tpu_sc as plsc

B = 16384
F = 26
V = 100000
D = 16
N_DENSE = 13
SP = F * D

NW = 32
ROWS_PER_W = SP // NW
OH = B // 2
UNROLL = 16


def _gather_body(idxT_hbm, table_hbm, out_hbm, idx_v, row_v, out_v):
    wid = lax.axis_index("s") * 2 + lax.axis_index("c")
    r0 = wid * ROWS_PER_W

    def do_row(j, f_prev):
        r = r0 + j
        f = r // D

        @pl.when(f != f_prev)
        def _stage_idx():
            pltpu.sync_copy(idxT_hbm.at[f], idx_v)

        pltpu.sync_copy(table_hbm.at[r], row_v)

        def do_half(h, carry2):
            @plsc.parallel_loop(0, OH, step=16, unroll=UNROLL)
            def gblk(i):
                iv = idx_v[pl.ds(h * OH + i, 16)]
                out_v[pl.ds(i, 16)] = plsc.load_gather(row_v, [iv])

            pltpu.sync_copy(out_v, out_hbm.at[r, pl.ds(h * OH, OH)])
            return carry2

        lax.fori_loop(0, 2, do_half, 0)
        return f

    lax.fori_loop(0, ROWS_PER_W, do_row, jnp.int32(-1))


@functools.cache
def _gather():
    return pl.kernel(
        _gather_body,
        out_type=jax.ShapeDtypeStruct((SP, B), jnp.float32),
        mesh=plsc.VectorSubcoreMesh(core_axis_name="c", subcore_axis_name="s"),
        scratch_types=[
            pltpu.VMEM((B,), jnp.int32),
            pltpu.VMEM((V,), jnp.float32),
            pltpu.VMEM((OH,), jnp.float32),
        ],
        compiler_params=pltpu.CompilerParams(needs_layout_passes=False),
    )


BB = 4096


def _tc_body(xs_ref, xd_ref, w1_ref, b1_ref, w2_ref, b2_ref,
             w3t_ref, b3_ref, wdt_ref, bd_ref, wfmt_ref, bfm_ref,
             o_ref):
    f32 = jnp.float32
    prec = lax.Precision.DEFAULT

    def dott(a, b):
        return lax.dot_general(a, b, (((0,), (0,)), ((), ())),
                               preferred_element_type=f32, precision=prec)

    def dotn(a, b):
        return lax.dot_general(a, b, (((1,), (0,)), ((), ())),
                               preferred_element_type=f32, precision=prec)

    xs = xs_ref[...]
    xd = xd_ref[...]
    ci = lax.broadcasted_iota(jnp.int32, (D, SP), 0)
    cj = lax.broadcasted_iota(jnp.int32, (D, SP), 1)
    sel = jnp.where((cj % D) == ci, 1.0, 0.0).astype(f32)
    s1 = dotn(sel, xs)
    s2 = dotn(sel, xs * xs)
    fm2 = 0.5 * jnp.sum(s1 * s1 - s2, axis=0, keepdims=True)
    wfmt = wfmt_ref[...]
    fm1 = dotn(wfmt[:, :SP], xs) + dotn(wfmt[:, SP:], xd) + bfm_ref[...]
    w1 = w1_ref[...]
    h = jnp.maximum(dott(w1[:SP], xs) + dott(w1[SP:], xd) + b1_ref[...], 0.0)
    h = jnp.maximum(dott(w2_ref[...], h) + b2_ref[...], 0.0)
    h = jnp.maximum(dotn(w3t_ref[...], h) + b3_ref[...], 0.0)
    dnn = dotn(wdt_ref[...], h) + bd_ref[...]
    o_ref[...] = jax.nn.sigmoid(fm1 + fm2 + dnn)


def _full(shape):
    return pl.BlockSpec(shape, lambda i: tuple(0 for _ in shape))


_tc_call = pl.pallas_call(
    _tc_body,
    grid=(B // BB,),
    in_specs=[
        pl.BlockSpec((SP, BB), lambda i: (0, i)),
        pl.BlockSpec((N_DENSE, BB), lambda i: (0, i)),
        _full((SP + N_DENSE, 256)),
        _full((256, 1)),
        _full((256, 128)),
        _full((128, 1)),
        _full((64, 128)),
        _full((64, 1)),
        _full((1, 64)),
        _full((1, 1)),
        _full((1, SP + N_DENSE)),
        _full((1, 1)),
    ],
    out_specs=pl.BlockSpec((1, BB), lambda i: (0, i)),
    out_shape=jax.ShapeDtypeStruct((1, B), jnp.float32),
)


def kernel(dense_input, sparse_input, embed_tables, W_fm, b_fm,
           W1, b1, W2, b2, W3, b3, Wd, bd):
    tableT = embed_tables.transpose(0, 2, 1).reshape(SP, V)
    idxT = sparse_input.T
    xsT = _gather()(idxT, tableT)
    outT = _tc_call(
        xsT, dense_input.T,
        W1, b1.reshape(-1, 1),
        W2, b2.reshape(-1, 1),
        W3.T, b3.reshape(-1, 1),
        Wd.T, bd.reshape(1, 1),
        W_fm.T, b_fm.reshape(1, 1),
    )
    return outT.reshape(B)
